# baseline (speedup 1.0000x reference)
import functools

import jax
import jax.numpy as jnp
from jax import lax
from jax.experimental import pallas as pl
from jax.experimental.pallas import tpu as pltpu
from jax.experimental.pallas import tpu_sc as plsc

NC = 2
NS = 16
LANES = 16
NW = NC * NS

F32 = jnp.float32


def _rsqrt16(v):
    i = plsc.bitcast(v, jnp.int32)
    i = 0x5F3759DF - lax.shift_right_logical(i, 1)
    y = plsc.bitcast(i, F32)
    for _ in range(3):
        y = y * (1.5 - 0.5 * v * y * y)
    return y


def _make_sc1(npad, nchunks, dh):
    rows = npad // NS
    mesh = plsc.VectorSubcoreMesh(
        core_axis_name="c", subcore_axis_name="s", num_cores=NC, num_subcores=NS)

    hrows = rows // 8
    nbig = nchunks

    @functools.partial(
        pl.kernel,
        out_type=[
            jax.ShapeDtypeStruct((NC, npad, dh), F32),
            jax.ShapeDtypeStruct((npad,), F32),
            jax.ShapeDtypeStruct((NC, npad, dh), F32),
        ],
        mesh=mesh,
        scratch_types=[
            pltpu.VMEM_SHARED((npad,), F32),
            pltpu.VMEM_SHARED((npad, dh), F32),
            pltpu.VMEM((nbig, 256), jnp.int32),
            pltpu.VMEM((nbig, 256), jnp.int32),
            pltpu.VMEM((rows,), F32),
            pltpu.VMEM((hrows, dh), F32),
            pltpu.VMEM((2, 256, dh), F32),
            pltpu.SemaphoreType.DMA,
            pltpu.SemaphoreType.DMA,
            pltpu.SemaphoreType.DMA,
        ],
        compiler_params=pltpu.CompilerParams(
            needs_layout_passes=False, use_tc_tiling_on_sc=False),
    )
    def sc1(x_hbm, src_t, dst_t, out_acc, out_dinv, xs_hbm,
            deg_sh, acc_sh, src_v, dst_v, vec1, xbuf, gbuf, gsem, ssem, dsem):
        c = lax.axis_index("c")
        s = lax.axis_index("s")
        row0 = s * rows

        nh = nbig // 2
        pltpu.sync_copy(src_t.at[s], src_v.at[pl.ds(0, nh)])
        pltpu.sync_copy(src_t.at[s + NS], src_v.at[pl.ds(nh, nh)])
        pltpu.sync_copy(dst_t.at[s], dst_v.at[pl.ds(0, nh)])
        pltpu.sync_copy(dst_t.at[s + NS], dst_v.at[pl.ds(nh, nh)])

        def fill_ones(i, _):
            vec1[pl.ds(i * LANES, LANES)] = jnp.full((LANES,), 1.0, F32)
            return _
        lax.fori_loop(0, rows // LANES, fill_ones, None)
        pltpu.sync_copy(vec1, deg_sh.at[pl.ds(row0, rows)])
        plsc.subcore_barrier()

        def deg_step(j, _):
            pltpu.async_copy(vec1.at[pl.ds(0, 256)], deg_sh.at[dst_v.at[j]],
                             dsem, add=True)
            @pl.when(j >= 8)
            def _():
                pltpu.make_async_copy(
                    vec1.at[pl.ds(0, 256)], deg_sh.at[dst_v.at[0]], dsem).wait()
            return _
        lax.fori_loop(0, nbig, deg_step, None)
        for _ in range(8):
            pltpu.make_async_copy(
                vec1.at[pl.ds(0, 256)], deg_sh.at[dst_v.at[0]], dsem).wait()
        plsc.subcore_barrier()

        pltpu.sync_copy(deg_sh.at[pl.ds(row0, rows)], vec1)

        def rsqrt_step(i, _):
            sl = pl.ds(i * LANES, LANES)
            vec1[sl] = _rsqrt16(vec1[sl])
            return _
        lax.fori_loop(0, rows // LANES, rsqrt_step, None)

        @pl.when(c == 0)
        def _():
            pltpu.sync_copy(vec1, out_dinv.at[pl.ds(row0, rows)])

        for part in range(rows // hrows):
            r0 = row0 + part * hrows
            pltpu.sync_copy(x_hbm.at[pl.ds(r0, hrows), pl.ds(c * dh, dh)], xbuf)

            def scale_rows(i, _):
                dv = vec1[pl.ds(part * hrows + i * LANES, LANES)]
                for rr in range(LANES):
                    r = i * LANES + rr
                    bv = jnp.full((LANES,), dv[rr], F32)
                    for k in range(dh // LANES):
                        xbuf[r, k * LANES:(k + 1) * LANES] = (
                            xbuf[r, k * LANES:(k + 1) * LANES] * bv)
                return _
            lax.fori_loop(0, hrows // LANES, scale_rows, None)
            pltpu.sync_copy(xbuf, xs_hbm.at[c, pl.ds(r0, hrows)])
            pltpu.sync_copy(xbuf, acc_sh.at[pl.ds(r0, hrows)])
        plsc.subcore_barrier()

        pltpu.async_copy(xs_hbm.at[c].at[src_v.at[0]], gbuf.at[0], gsem)

        def agg_step(j, _):
            pltpu.make_async_copy(
                xs_hbm.at[c].at[src_v.at[j]], gbuf.at[j % 2], gsem).wait()
            pltpu.async_copy(gbuf.at[j % 2], acc_sh.at[dst_v.at[j]],
                             ssem, add=True)
            @pl.when(j >= 1)
            def _():
                pltpu.make_async_copy(
                    gbuf.at[0], acc_sh.at[dst_v.at[0]], ssem).wait()
            @pl.when(j + 1 < nbig)
            def _():
                pltpu.async_copy(xs_hbm.at[c].at[src_v.at[j + 1]],
                                 gbuf.at[(j + 1) % 2], gsem)
            return _
        lax.fori_loop(0, nbig, agg_step, None)
        pltpu.make_async_copy(gbuf.at[0], acc_sh.at[dst_v.at[0]], ssem).wait()
        plsc.subcore_barrier()

        pltpu.sync_copy(acc_sh.at[pl.ds(row0, rows)],
                        out_acc.at[c, pl.ds(row0, rows)])

    return sc1


def _make_sc2(npad, nchunks, dt):
    rows = npad // NS
    mesh = plsc.VectorSubcoreMesh(
        core_axis_name="c", subcore_axis_name="s", num_cores=NC, num_subcores=NS)

    @functools.partial(
        pl.kernel,
        out_type=jax.ShapeDtypeStruct((NC, npad, dt), F32),
        mesh=mesh,
        scratch_types=[
            pltpu.VMEM_SHARED((npad, dt), F32),
            pltpu.VMEM_SHARED((npad, dt), F32),
            pltpu.VMEM((nchunks, 128), jnp.int32),
            pltpu.VMEM((nchunks, 128), jnp.int32),
            pltpu.VMEM((rows, dt), F32),
            pltpu.VMEM((4, 128, dt), F32),
            pltpu.SemaphoreType.DMA,
            pltpu.SemaphoreType.DMA,
        ],
        compiler_params=pltpu.CompilerParams(
            needs_layout_passes=False, use_tc_tiling_on_sc=False),
    )
    def sc2(ts_hbm, src_t, dst_t, out, ts_sh, acc_sh, src_v, dst_v, tbuf, gbuf,
            gsem, ssem):
        c = lax.axis_index("c")
        s = lax.axis_index("s")
        w = c * NS + s
        row0 = s * rows

        pltpu.sync_copy(src_t.at[w], src_v)
        pltpu.sync_copy(dst_t.at[w], dst_v)

        pltpu.sync_copy(ts_hbm.at[pl.ds(row0, rows)], tbuf)
        pltpu.sync_copy(tbuf, ts_sh.at[pl.ds(row0, rows)])

        @pl.when(c == 1)
        def _():
            def zero_row(r, _):
                for k in range(dt // LANES):
                    tbuf[r, k * LANES:(k + 1) * LANES] = jnp.zeros((LANES,), F32)
                return _
            lax.fori_loop(0, rows, zero_row, None)
        pltpu.sync_copy(tbuf, acc_sh.at[pl.ds(row0, rows)])
        plsc.subcore_barrier()

        for jj in range(3):
            pltpu.async_copy(ts_sh.at[src_v.at[jj]], gbuf.at[jj], gsem)

        def agg_step(j, _):
            pltpu.make_async_copy(
                ts_sh.at[src_v.at[j]], gbuf.at[j % 4], gsem).wait()
            pltpu.async_copy(gbuf.at[j % 4], acc_sh.at[dst_v.at[j]], ssem, add=True)
            @pl.when(j >= 1)
            def _():
                pltpu.make_async_copy(
                    gbuf.at[0], acc_sh.at[dst_v.at[0]], ssem).wait()
            @pl.when(j + 3 < nchunks)
            def _():
                pltpu.async_copy(ts_sh.at[src_v.at[j + 3]],
                                 gbuf.at[(j + 3) % 4], gsem)
            return _
        lax.fori_loop(0, nchunks, agg_step, None)
        pltpu.make_async_copy(gbuf.at[0], acc_sh.at[dst_v.at[0]], ssem).wait()
        plsc.subcore_barrier()

        pltpu.sync_copy(acc_sh.at[pl.ds(row0, rows)], out.at[c, pl.ds(row0, rows)])

    return sc2


def _tc1_body(alo, ahi, dinv, w1, b1, wmu, wh, out, w2s):
    hi = jax.lax.Precision.HIGHEST
    @pl.when(pl.program_id(0) == 0)
    def _():
        w2s[...] = jnp.dot(wmu[...], wh[...], preferred_element_type=F32,
                           precision=hi)
    agg = jnp.concatenate([alo[0], ahi[0]], axis=1) * dinv[...]
    h = jnp.maximum(
        jnp.dot(agg, w1[...], preferred_element_type=F32) + b1[...], 0.0)
    w2 = w2s[...]
    t0 = jnp.sum(h * w2[:, 0][None, :], axis=1, keepdims=True)
    t1 = jnp.sum(h * w2[:, 1][None, :], axis=1, keepdims=True)
    t = jnp.concatenate([t0, t1], axis=1)
    ts = t * dinv[...]
    out[...] = jnp.concatenate(
        [ts, jnp.zeros((ts.shape[0], 14), F32)], axis=1)


def _tc2_body(a0, a1, dinv, bmu, wh, bh, out):
    c2 = jnp.dot(bmu[...], wh[...], preferred_element_type=F32) + bh[...]
    out[...] = (a0[0] + a1[0])[:, 0:2] * dinv[...] + c2


def kernel(x, edge_index, W1, b1, Wmu, bmu, Wls, bls, Wh, bh):
    n, d_in = x.shape
    e = edge_index.shape[1]
    hid = W1.shape[1]
    inner = Wmu.shape[1]
    ncls = Wh.shape[1]
    dh = d_in // 2
    npad = ((n + NW * LANES - 1) // (NW * LANES)) * (NW * LANES)
    e_per_w = (((e + NW - 1) // NW + 255) // 256) * 256
    nchunks = e_per_w // 128
    epad = e_per_w * NW

    pad = jnp.full((epad - e,), n, jnp.int32)
    src_flat = jnp.concatenate([edge_index[0], pad])
    dst_flat = jnp.concatenate([edge_index[1], pad])
    src_t = src_flat.reshape(NW, nchunks, 128)
    dst_t = dst_flat.reshape(NW, nchunks, 128)
    src_t2 = src_flat.reshape(NW, nchunks // 2, 256)
    dst_t2 = dst_flat.reshape(NW, nchunks // 2, 256)
    x_p = jnp.pad(x, ((0, npad - n), (0, 0)))

    acc, dinv, _xs = _make_sc1(npad, nchunks, dh)(x_p, src_t2, dst_t2)
    dinv2 = dinv.reshape(npad, 1)

    bn = 1024
    grid = npad // bn
    ts16 = pl.pallas_call(
        _tc1_body,
        grid=(grid,),
        in_specs=[
            pl.BlockSpec((1, bn, dh), lambda i: (0, i, 0)),
            pl.BlockSpec((1, bn, dh), lambda i: (1, i, 0)),
            pl.BlockSpec((bn, 1), lambda i: (i, 0)),
            pl.BlockSpec((d_in, hid), lambda i: (0, 0)),
            pl.BlockSpec((1, hid), lambda i: (0, 0)),
            pl.BlockSpec((hid, inner), lambda i: (0, 0)),
            pl.BlockSpec((inner, ncls), lambda i: (0, 0)),
        ],
        out_specs=pl.BlockSpec((bn, 16), lambda i: (i, 0)),
        out_shape=jax.ShapeDtypeStruct((npad, 16), F32),
        scratch_shapes=[pltpu.VMEM((hid, ncls), F32)],
    )(acc, acc, dinv2, W1, b1.reshape(1, hid), Wmu, Wh)

    acc2 = _make_sc2(npad, nchunks, 16)(ts16, src_t, dst_t)

    logits = pl.pallas_call(
        _tc2_body,
        grid=(grid,),
        in_specs=[
            pl.BlockSpec((1, bn, 16), lambda i: (0, i, 0)),
            pl.BlockSpec((1, bn, 16), lambda i: (1, i, 0)),
            pl.BlockSpec((bn, 1), lambda i: (i, 0)),
            pl.BlockSpec((1, inner), lambda i: (0, 0)),
            pl.BlockSpec((inner, ncls), lambda i: (0, 0)),
            pl.BlockSpec((1, ncls), lambda i: (0, 0)),
        ],
        out_specs=pl.BlockSpec((bn, ncls), lambda i: (i, 0)),
        out_shape=jax.ShapeDtypeStruct((npad, ncls), F32),
    )(acc2, acc2, dinv2, bmu.reshape(1, inner), Wh, bh.reshape(1, ncls))

    return logits[:n]


# speedup vs baseline: 1.4536x; 1.4536x over previous
import functools

import jax
import jax.numpy as jnp
from jax import lax
from jax.experimental import pallas as pl
from jax.experimental.pallas import tpu as pltpu
from jax.experimental.pallas import tpu_sc as plsc

NC = 2
NS = 16
LANES = 16
NW = NC * NS

F32 = jnp.float32


def _rsqrt16(v):
    i = plsc.bitcast(v, jnp.int32)
    i = 0x5F3759DF - lax.shift_right_logical(i, 1)
    y = plsc.bitcast(i, F32)
    for _ in range(3):
        y = y * (1.5 - 0.5 * v * y * y)
    return y


def _make_sc1(npad, nchunks, dh):
    rows = npad // NS
    mesh = plsc.VectorSubcoreMesh(
        core_axis_name="c", subcore_axis_name="s", num_cores=NC, num_subcores=NS)

    hrows = rows // 8
    nch2 = 2 * nchunks
    DEPTH = 4

    @functools.partial(
        pl.kernel,
        out_type=[
            jax.ShapeDtypeStruct((NC, npad, dh), F32),
            jax.ShapeDtypeStruct((npad,), F32),
            jax.ShapeDtypeStruct((NC, npad, dh), F32),
        ],
        mesh=mesh,
        scratch_types=[
            pltpu.VMEM_SHARED((npad,), F32),
            pltpu.VMEM_SHARED((npad, dh), F32),
            pltpu.VMEM((nch2, 128), jnp.int32),
            pltpu.VMEM((nch2, 128), jnp.int32),
            pltpu.VMEM((rows,), F32),
            pltpu.VMEM((hrows, dh), F32),
            pltpu.VMEM((DEPTH, 128, dh), F32),
            pltpu.SemaphoreType.DMA,
            pltpu.SemaphoreType.DMA,
            pltpu.SemaphoreType.DMA,
        ],
        compiler_params=pltpu.CompilerParams(
            needs_layout_passes=False, use_tc_tiling_on_sc=False),
    )
    def sc1(x_hbm, src_t, dst_t, out_acc, out_dinv, xs_hbm,
            deg_sh, acc_sh, src_v, dst_v, vec1, xbuf, gbuf, gsem, ssem, dsem):
        c = lax.axis_index("c")
        s = lax.axis_index("s")
        row0 = s * rows

        pltpu.sync_copy(src_t.at[s], src_v.at[pl.ds(0, nchunks)])
        pltpu.sync_copy(src_t.at[s + NS], src_v.at[pl.ds(nchunks, nchunks)])
        pltpu.sync_copy(dst_t.at[s], dst_v.at[pl.ds(0, nchunks)])
        pltpu.sync_copy(dst_t.at[s + NS], dst_v.at[pl.ds(nchunks, nchunks)])

        def fill_ones(i, _):
            vec1[pl.ds(i * LANES, LANES)] = jnp.full((LANES,), 1.0, F32)
            return _
        lax.fori_loop(0, rows // LANES, fill_ones, None)
        pltpu.sync_copy(vec1, deg_sh.at[pl.ds(row0, rows)])
        plsc.subcore_barrier()

        def deg_step(j, _):
            pltpu.async_copy(vec1.at[pl.ds(0, 128)], deg_sh.at[dst_v.at[j]],
                             dsem, add=True)
            @pl.when(j >= 16)
            def _():
                pltpu.make_async_copy(
                    vec1.at[pl.ds(0, 128)], deg_sh.at[dst_v.at[0]], dsem).wait()
            return _
        lax.fori_loop(0, nch2, deg_step, None)
        for _ in range(16):
            pltpu.make_async_copy(
                vec1.at[pl.ds(0, 128)], deg_sh.at[dst_v.at[0]], dsem).wait()
        plsc.subcore_barrier()

        pltpu.sync_copy(deg_sh.at[pl.ds(row0, rows)], vec1)

        def rsqrt_step(i, _):
            sl = pl.ds(i * LANES, LANES)
            vec1[sl] = _rsqrt16(vec1[sl])
            return _
        lax.fori_loop(0, rows // LANES, rsqrt_step, None)

        @pl.when(c == 0)
        def _():
            pltpu.sync_copy(vec1, out_dinv.at[pl.ds(row0, rows)])

        for part in range(rows // hrows):
            r0 = row0 + part * hrows
            pltpu.sync_copy(x_hbm.at[pl.ds(r0, hrows), pl.ds(c * dh, dh)], xbuf)

            def scale_rows(i, _):
                dv = vec1[pl.ds(part * hrows + i * LANES, LANES)]
                for rr in range(LANES):
                    r = i * LANES + rr
                    bv = jnp.full((LANES,), dv[rr], F32)
                    for k in range(dh // LANES):
                        xbuf[r, k * LANES:(k + 1) * LANES] = (
                            xbuf[r, k * LANES:(k + 1) * LANES] * bv)
                return _
            lax.fori_loop(0, hrows // LANES, scale_rows, None)
            pltpu.sync_copy(xbuf, xs_hbm.at[c, pl.ds(r0, hrows)])
            pltpu.sync_copy(xbuf, acc_sh.at[pl.ds(r0, hrows)])
        plsc.subcore_barrier()

        for jj in range(DEPTH - 1):
            pltpu.async_copy(xs_hbm.at[c].at[src_v.at[jj]], gbuf.at[jj], gsem)

        def agg_step(j, _):
            pltpu.make_async_copy(
                xs_hbm.at[c].at[src_v.at[j]], gbuf.at[j % DEPTH], gsem).wait()
            pltpu.async_copy(gbuf.at[j % DEPTH], acc_sh.at[dst_v.at[j]],
                             ssem, add=True)
            @pl.when(j >= 1)
            def _():
                pltpu.make_async_copy(
                    gbuf.at[0], acc_sh.at[dst_v.at[0]], ssem).wait()
            @pl.when(j + DEPTH - 1 < nch2)
            def _():
                pltpu.async_copy(xs_hbm.at[c].at[src_v.at[j + DEPTH - 1]],
                                 gbuf.at[(j + DEPTH - 1) % DEPTH], gsem)
            return _
        lax.fori_loop(0, nch2, agg_step, None)
        pltpu.make_async_copy(gbuf.at[0], acc_sh.at[dst_v.at[0]], ssem).wait()
        plsc.subcore_barrier()

        pltpu.sync_copy(acc_sh.at[pl.ds(row0, rows)],
                        out_acc.at[c, pl.ds(row0, rows)])

    return sc1


def _make_sc2(npad, nchunks, dt):
    rows = npad // NS
    mesh = plsc.VectorSubcoreMesh(
        core_axis_name="c", subcore_axis_name="s", num_cores=NC, num_subcores=NS)

    @functools.partial(
        pl.kernel,
        out_type=jax.ShapeDtypeStruct((NC, npad, dt), F32),
        mesh=mesh,
        scratch_types=[
            pltpu.VMEM_SHARED((npad, dt), F32),
            pltpu.VMEM_SHARED((npad, dt), F32),
            pltpu.VMEM((nchunks, 128), jnp.int32),
            pltpu.VMEM((nchunks, 128), jnp.int32),
            pltpu.VMEM((rows, dt), F32),
            pltpu.VMEM((4, 128, dt), F32),
            pltpu.SemaphoreType.DMA,
            pltpu.SemaphoreType.DMA,
        ],
        compiler_params=pltpu.CompilerParams(
            needs_layout_passes=False, use_tc_tiling_on_sc=False),
    )
    def sc2(ts_hbm, src_t, dst_t, out, ts_sh, acc_sh, src_v, dst_v, tbuf, gbuf,
            gsem, ssem):
        c = lax.axis_index("c")
        s = lax.axis_index("s")
        w = c * NS + s
        row0 = s * rows

        pltpu.sync_copy(src_t.at[w], src_v)
        pltpu.sync_copy(dst_t.at[w], dst_v)

        pltpu.sync_copy(ts_hbm.at[pl.ds(row0, rows)], tbuf)
        pltpu.sync_copy(tbuf, ts_sh.at[pl.ds(row0, rows)])

        @pl.when(c == 1)
        def _():
            def zero_row(r, _):
                for k in range(dt // LANES):
                    tbuf[r, k * LANES:(k + 1) * LANES] = jnp.zeros((LANES,), F32)
                return _
            lax.fori_loop(0, rows, zero_row, None)
        pltpu.sync_copy(tbuf, acc_sh.at[pl.ds(row0, rows)])
        plsc.subcore_barrier()

        for jj in range(3):
            pltpu.async_copy(ts_sh.at[src_v.at[jj]], gbuf.at[jj], gsem)

        def agg_step(j, _):
            pltpu.make_async_copy(
                ts_sh.at[src_v.at[j]], gbuf.at[j % 4], gsem).wait()
            pltpu.async_copy(gbuf.at[j % 4], acc_sh.at[dst_v.at[j]], ssem, add=True)
            @pl.when(j >= 1)
            def _():
                pltpu.make_async_copy(
                    gbuf.at[0], acc_sh.at[dst_v.at[0]], ssem).wait()
            @pl.when(j + 3 < nchunks)
            def _():
                pltpu.async_copy(ts_sh.at[src_v.at[j + 3]],
                                 gbuf.at[(j + 3) % 4], gsem)
            return _
        lax.fori_loop(0, nchunks, agg_step, None)
        pltpu.make_async_copy(gbuf.at[0], acc_sh.at[dst_v.at[0]], ssem).wait()
        plsc.subcore_barrier()

        pltpu.sync_copy(acc_sh.at[pl.ds(row0, rows)], out.at[c, pl.ds(row0, rows)])

    return sc2


def _tc1_body(alo, ahi, dinv, w1, b1, wmu, wh, out, w2s):
    hi = jax.lax.Precision.HIGHEST
    @pl.when(pl.program_id(0) == 0)
    def _():
        w2s[...] = jnp.dot(wmu[...], wh[...], preferred_element_type=F32,
                           precision=hi)
    agg = jnp.concatenate([alo[0], ahi[0]], axis=1) * dinv[...]
    h = jnp.maximum(
        jnp.dot(agg, w1[...], preferred_element_type=F32) + b1[...], 0.0)
    w2 = w2s[...]
    t0 = jnp.sum(h * w2[:, 0][None, :], axis=1, keepdims=True)
    t1 = jnp.sum(h * w2[:, 1][None, :], axis=1, keepdims=True)
    t = jnp.concatenate([t0, t1], axis=1)
    ts = t * dinv[...]
    out[...] = jnp.concatenate(
        [ts, jnp.zeros((ts.shape[0], 14), F32)], axis=1)


def _tc2_body(a0, a1, dinv, bmu, wh, bh, out):
    c2 = jnp.dot(bmu[...], wh[...], preferred_element_type=F32) + bh[...]
    out[...] = (a0[0] + a1[0])[:, 0:2] * dinv[...] + c2


def kernel(x, edge_index, W1, b1, Wmu, bmu, Wls, bls, Wh, bh):
    n, d_in = x.shape
    e = edge_index.shape[1]
    hid = W1.shape[1]
    inner = Wmu.shape[1]
    ncls = Wh.shape[1]
    dh = d_in // 2
    npad = ((n + NW * LANES - 1) // (NW * LANES)) * (NW * LANES)
    e_per_w = (((e + NW - 1) // NW + 127) // 128) * 128
    nchunks = e_per_w // 128
    epad = e_per_w * NW

    pad = jnp.full((epad - e,), n, jnp.int32)
    src_t = jnp.concatenate([edge_index[0], pad]).reshape(NW, nchunks, 128)
    dst_t = jnp.concatenate([edge_index[1], pad]).reshape(NW, nchunks, 128)
    x_p = jnp.pad(x, ((0, npad - n), (0, 0)))

    acc, dinv, _xs = _make_sc1(npad, nchunks, dh)(x_p, src_t, dst_t)
    dinv2 = dinv.reshape(npad, 1)

    bn = 1024
    grid = npad // bn
    ts16 = pl.pallas_call(
        _tc1_body,
        grid=(grid,),
        in_specs=[
            pl.BlockSpec((1, bn, dh), lambda i: (0, i, 0)),
            pl.BlockSpec((1, bn, dh), lambda i: (1, i, 0)),
            pl.BlockSpec((bn, 1), lambda i: (i, 0)),
            pl.BlockSpec((d_in, hid), lambda i: (0, 0)),
            pl.BlockSpec((1, hid), lambda i: (0, 0)),
            pl.BlockSpec((hid, inner), lambda i: (0, 0)),
            pl.BlockSpec((inner, ncls), lambda i: (0, 0)),
        ],
        out_specs=pl.BlockSpec((bn, 16), lambda i: (i, 0)),
        out_shape=jax.ShapeDtypeStruct((npad, 16), F32),
        scratch_shapes=[pltpu.VMEM((hid, ncls), F32)],
    )(acc, acc, dinv2, W1, b1.reshape(1, hid), Wmu, Wh)

    acc2 = _make_sc2(npad, nchunks, 16)(ts16, src_t, dst_t)

    logits = pl.pallas_call(
        _tc2_body,
        grid=(grid,),
        in_specs=[
            pl.BlockSpec((1, bn, 16), lambda i: (0, i, 0)),
            pl.BlockSpec((1, bn, 16), lambda i: (1, i, 0)),
            pl.BlockSpec((bn, 1), lambda i: (i, 0)),
            pl.BlockSpec((1, inner), lambda i: (0, 0)),
            pl.BlockSpec((inner, ncls), lambda i: (0, 0)),
            pl.BlockSpec((1, ncls), lambda i: (0, 0)),
        ],
        out_specs=pl.BlockSpec((bn, ncls), lambda i: (i, 0)),
        out_shape=jax.ShapeDtypeStruct((npad, ncls), F32),
    )(acc2, acc2, dinv2, bmu.reshape(1, inner), Wh, bh.reshape(1, ncls))

    return logits[:n]
